# baseline (speedup 1.0000x reference)
import jax
import jax.numpy as jnp
from jax import lax
from jax.experimental import pallas as pl
from jax.experimental.pallas import tpu as pltpu
from jax.experimental.pallas import tpu_sc as plsc

_SEQ = 200
_D = 64
_L = 16
_NC = 2
_NS = 16
_NW = _NC * _NS
_BB = 128
_NBUF = 4
_SCALE = 8.0


def _body(idx_hbm, table_hbm, pos_hbm, out_hbm,
          idx_blk, idx_t, rows_v, tile0, tile1, tile2, tile3, pos_v,
          gsems, osems):
    tiles = (tile0, tile1, tile2, tile3)
    wid = lax.axis_index("s") * _NC + lax.axis_index("c")

    lanes = lax.iota(jnp.int32, _L)
    zeros = lanes * 0
    row_idx = [lanes + blk * _L for blk in range(_BB // _L)]

    pltpu.sync_copy(pos_hbm, pos_v)

    half = _BB // 2
    for piece in range(2):
        pltpu.sync_copy(idx_hbm.at[pl.ds(wid * _BB + piece * half, half)],
                        idx_blk)

        @plsc.parallel_loop(0, _SEQ, step=1, unroll=2)
        def _(s):
            for blk in range(half // _L):
                idx_t[s, pl.ds(piece * half + blk * _L, _L)] = (
                    plsc.load_gather(idx_blk, [row_idx[blk], zeros + s]))

    def gather_start(slot, s):
        pltpu.async_copy(table_hbm.at[idx_t.at[s]], rows_v.at[slot],
                         gsems.at[slot])

    def gather_wait(slot):
        pltpu.make_async_copy(table_hbm.at[idx_t.at[0]], rows_v.at[slot],
                              gsems.at[slot]).wait()

    def out_start(slot, s):
        for ct in range(_D // 8):
            pltpu.async_copy(
                tiles[ct // 2].at[slot, pl.ds((ct % 2) * 1024, 1024)],
                out_hbm.at[s, ct, wid], osems.at[slot])

    def out_wait(slot):
        for ct in range(_D // 8):
            pltpu.make_async_copy(
                tiles[ct // 2].at[slot, pl.ds((ct % 2) * 1024, 1024)],
                out_hbm.at[0, ct, wid], osems.at[slot]).wait()

    for b in range(_NBUF):
        gather_start(b, b)

    def ring(m, carry):
        for b in range(_NBUF):
            s = _NBUF * m + b

            gather_wait(b)

            @pl.when(s >= _NBUF)
            def _():
                out_wait(b)

            rows = rows_v.at[b]
            btiles = [t.at[b] for t in tiles]

            @plsc.parallel_loop(0, _L, step=1, unroll=1)
            def _(d):
                perm = (lanes + d) & (_L - 1)
                perm128 = perm << 7
                for cb in range(_D // _L):
                    col = perm + cb * _L
                    pb = plsc.load_gather(pos_v, [zeros + s, col])
                    for rb in range(_BB // _L):
                        g = plsc.load_gather(rows, [row_idx[rb], col])
                        plsc.store_scatter(
                            btiles[cb], [perm128 + row_idx[rb]],
                            g * _SCALE + pb)

            out_start(b, s)

            @pl.when(s + _NBUF < _SEQ)
            def _():
                gather_start(b, s + _NBUF)
        return carry

    lax.fori_loop(0, _SEQ // _NBUF, ring, 0)
    for b in range(_NBUF):
        out_wait(b)


def kernel(inputs, token_table, pos_table):
    b, s = inputs.shape
    _, d = token_table.shape
    mesh = plsc.VectorSubcoreMesh(
        core_axis_name="c", subcore_axis_name="s",
        num_cores=_NC, num_subcores=_NS,
    )
    out5 = pl.kernel(
        _body,
        out_type=jax.ShapeDtypeStruct((s, d // 8, b // _BB, 8 * _BB),
                                      jnp.float32),
        mesh=mesh,
        compiler_params=pltpu.CompilerParams(use_tc_tiling_on_sc=False,
                                             needs_layout_passes=False),
        scratch_types=[
            pltpu.VMEM((_BB // 2, _SEQ), jnp.int32),
            pltpu.VMEM((_SEQ, _BB), jnp.int32),
            pltpu.VMEM((_NBUF, _BB, _D), jnp.float32),
            pltpu.VMEM((_NBUF, _L * _BB), jnp.float32),
            pltpu.VMEM((_NBUF, _L * _BB), jnp.float32),
            pltpu.VMEM((_NBUF, _L * _BB), jnp.float32),
            pltpu.VMEM((_NBUF, _L * _BB), jnp.float32),
            pltpu.VMEM((_SEQ, _D), jnp.float32),
            pltpu.SemaphoreType.DMA((_NBUF,)),
            pltpu.SemaphoreType.DMA((_NBUF,)),
        ],
    )(inputs, token_table, pos_table)
    out5 = out5.reshape(s, d // 8, b // _BB, 8, _BB)
    return out5.transpose(2, 4, 0, 1, 3).reshape(b, s, d)


# speedup vs baseline: 1.1011x; 1.1011x over previous
import jax
import jax.numpy as jnp
from jax import lax
from jax.experimental import pallas as pl
from jax.experimental.pallas import tpu as pltpu
from jax.experimental.pallas import tpu_sc as plsc

_SEQ = 200
_D = 64
_L = 16
_NC = 2
_NS = 16
_NW = _NC * _NS
_BB = 128
_NBUF = 4
_SCALE = 8.0


def _body(idx_hbm, table_hbm, pos_hbm, out_hbm,
          idx_blk, idx_t, rows_v, tile_v, pos_v, gsems, osems):
    wid = lax.axis_index("s") * _NC + lax.axis_index("c")

    lanes = lax.iota(jnp.int32, _L)
    zeros = lanes * 0
    row_idx = [lanes + blk * _L for blk in range(_BB // _L)]

    pltpu.sync_copy(pos_hbm, pos_v)

    half = _BB // 2
    for piece in range(2):
        pltpu.sync_copy(idx_hbm.at[pl.ds(wid * _BB + piece * half, half)],
                        idx_blk)

        @plsc.parallel_loop(0, _SEQ, step=1, unroll=2)
        def _(s):
            for blk in range(half // _L):
                idx_t[s, pl.ds(piece * half + blk * _L, _L)] = (
                    plsc.load_gather(idx_blk, [row_idx[blk], zeros + s]))

    def gather_start(slot, s):
        pltpu.async_copy(table_hbm.at[idx_t.at[s]], rows_v.at[slot],
                         gsems.at[slot])

    def gather_wait(slot):
        pltpu.make_async_copy(table_hbm.at[idx_t.at[0]], rows_v.at[slot],
                              gsems.at[slot]).wait()

    def out_start(slot, s):
        for ct in range(_D // 8):
            pltpu.async_copy(tile_v.at[slot, pl.ds(ct * 1024, 1024)],
                             out_hbm.at[s, ct, wid], osems.at[slot])

    def out_wait(slot):
        for ct in range(_D // 8):
            pltpu.make_async_copy(tile_v.at[slot, pl.ds(ct * 1024, 1024)],
                                  out_hbm.at[0, ct, wid],
                                  osems.at[slot]).wait()

    for b in range(_NBUF):
        gather_start(b, b)

    def ring(m, carry):
        for b in range(_NBUF):
            s = _NBUF * m + b

            gather_wait(b)

            @pl.when(s >= _NBUF)
            def _():
                out_wait(b)

            rows = rows_v.at[b]
            tile = tile_v.at[b]

            @plsc.parallel_loop(0, _L, step=1, unroll=1)
            def _(d):
                perm = (lanes + d) & (_L - 1)
                perm128 = perm << 7
                for cb in range(_D // _L):
                    col = perm + cb * _L
                    st_base = perm128 + cb * (_L * _BB)
                    pb = plsc.load_gather(pos_v, [zeros + s, col])
                    for rb in range(_BB // _L):
                        g = plsc.load_gather(rows, [row_idx[rb], col])
                        plsc.store_scatter(
                            tile, [st_base + row_idx[rb]],
                            g * _SCALE + pb)

            out_start(b, s)

            @pl.when(s + _NBUF < _SEQ)
            def _():
                gather_start(b, s + _NBUF)
        return carry

    lax.fori_loop(0, _SEQ // _NBUF, ring, 0)
    for b in range(_NBUF):
        out_wait(b)


def kernel(inputs, token_table, pos_table):
    b, s = inputs.shape
    _, d = token_table.shape
    mesh = plsc.VectorSubcoreMesh(
        core_axis_name="c", subcore_axis_name="s",
        num_cores=_NC, num_subcores=_NS,
    )
    out5 = pl.kernel(
        _body,
        out_type=jax.ShapeDtypeStruct((s, d // 8, b // _BB, 8 * _BB),
                                      jnp.float32),
        mesh=mesh,
        compiler_params=pltpu.CompilerParams(use_tc_tiling_on_sc=False,
                                             needs_layout_passes=False),
        scratch_types=[
            pltpu.VMEM((_BB // 2, _SEQ), jnp.int32),
            pltpu.VMEM((_SEQ, _BB), jnp.int32),
            pltpu.VMEM((_NBUF, _BB, _D), jnp.float32),
            pltpu.VMEM((_NBUF, _D * _BB), jnp.float32),
            pltpu.VMEM((_SEQ, _D), jnp.float32),
            pltpu.SemaphoreType.DMA((_NBUF,)),
            pltpu.SemaphoreType.DMA((_NBUF,)),
        ],
    )(inputs, token_table, pos_table)
    out5 = out5.reshape(s, d // 8, b // _BB, 8, _BB)
    return out5.transpose(2, 4, 0, 1, 3).reshape(b, s, d)


# speedup vs baseline: 1.1747x; 1.0669x over previous
import jax
import jax.numpy as jnp
from jax import lax
from jax.experimental import pallas as pl
from jax.experimental.pallas import tpu as pltpu
from jax.experimental.pallas import tpu_sc as plsc

_SEQ = 200
_D = 64
_L = 16
_NC = 2
_NS = 16
_NW = _NC * _NS
_BB = 128
_NBUF = 4
_SCALE = 8.0


def _body(idx_hbm, table_hbm, pos_hbm, out_hbm,
          idx_blk, idx_t, rows_v, tile_v, pos_v, gsems, osems):
    wid = lax.axis_index("s") * _NC + lax.axis_index("c")

    lanes = lax.iota(jnp.int32, _L)
    zeros = lanes * 0
    row_idx = [lanes + blk * _L for blk in range(_BB // _L)]

    pltpu.sync_copy(pos_hbm, pos_v)

    half = _BB // 2
    for piece in range(2):
        pltpu.sync_copy(idx_hbm.at[pl.ds(wid * _BB + piece * half, half)],
                        idx_blk)

        @plsc.parallel_loop(0, _SEQ, step=1, unroll=2)
        def _(s):
            for blk in range(half // _L):
                idx_t[s, pl.ds(piece * half + blk * _L, _L)] = (
                    plsc.load_gather(idx_blk, [row_idx[blk], zeros + s]))

    def gather_start(slot, s):
        pltpu.async_copy(table_hbm.at[idx_t.at[s]], rows_v.at[slot],
                         gsems.at[slot])

    def gather_wait(slot):
        pltpu.make_async_copy(table_hbm.at[idx_t.at[0]], rows_v.at[slot],
                              gsems.at[slot]).wait()

    def out_start(slot, s):
        for ct in range(_D // 8):
            pltpu.async_copy(tile_v.at[slot, pl.ds(ct * 1024, 1024)],
                             out_hbm.at[s, ct, wid], osems.at[slot])

    def out_wait(slot):
        for ct in range(_D // 8):
            pltpu.make_async_copy(tile_v.at[slot, pl.ds(ct * 1024, 1024)],
                                  out_hbm.at[0, ct, wid],
                                  osems.at[slot]).wait()

    for b in range(_NBUF):
        gather_start(b, b)

    def ring(m, carry):
        for b in range(_NBUF):
            s = _NBUF * m + b

            gather_wait(b)

            @pl.when(s >= _NBUF)
            def _():
                out_wait(b)

            rows = rows_v.at[b]
            tile = tile_v.at[b]

            @plsc.parallel_loop(0, _L, step=1, unroll=2)
            def _(d):
                perm = (lanes + d) & (_L - 1)
                perm128 = perm << 7
                for cb in range(_D // _L):
                    col = perm + cb * _L
                    st_base = perm128 + cb * (_L * _BB)
                    pb = plsc.load_gather(pos_v, [zeros + s, col])
                    for rb in range(_BB // _L):
                        g = plsc.load_gather(rows, [row_idx[rb], col])
                        plsc.store_scatter(
                            tile, [st_base + row_idx[rb]],
                            g * _SCALE + pb)

            out_start(b, s)

            @pl.when(s + _NBUF < _SEQ)
            def _():
                gather_start(b, s + _NBUF)
        return carry

    lax.fori_loop(0, _SEQ // _NBUF, ring, 0)
    for b in range(_NBUF):
        out_wait(b)


def kernel(inputs, token_table, pos_table):
    b, s = inputs.shape
    _, d = token_table.shape
    mesh = plsc.VectorSubcoreMesh(
        core_axis_name="c", subcore_axis_name="s",
        num_cores=_NC, num_subcores=_NS,
    )
    out5 = pl.kernel(
        _body,
        out_type=jax.ShapeDtypeStruct((s, d // 8, b // _BB, 8 * _BB),
                                      jnp.float32),
        mesh=mesh,
        compiler_params=pltpu.CompilerParams(use_tc_tiling_on_sc=False,
                                             needs_layout_passes=False),
        scratch_types=[
            pltpu.VMEM((_BB // 2, _SEQ), jnp.int32),
            pltpu.VMEM((_SEQ, _BB), jnp.int32),
            pltpu.VMEM((_NBUF, _BB, _D), jnp.float32),
            pltpu.VMEM((_NBUF, _D * _BB), jnp.float32),
            pltpu.VMEM((_SEQ, _D), jnp.float32),
            pltpu.SemaphoreType.DMA((_NBUF,)),
            pltpu.SemaphoreType.DMA((_NBUF,)),
        ],
    )(inputs, token_table, pos_table)
    out5 = out5.reshape(s, d // 8, b // _BB, 8, _BB)
    return out5.transpose(2, 4, 0, 1, 3).reshape(b, s, d)
